# Initial kernel scaffold; baseline (speedup 1.0000x reference)
import jax
import jax.numpy as jnp
from jax.experimental import pallas as pl


def kernel(x, linear_w, emb_w, bias):
    raise NotImplementedError("write your pallas kernel here")


# speedup vs baseline: 1.3573x; 1.3573x over previous
import functools

import jax
import jax.numpy as jnp
from jax import lax
from jax.experimental import pallas as pl
from jax.experimental.pallas import tpu as pltpu
from jax.experimental.pallas import tpu_sc as plsc

_FIELD = 38461
_F = 26
_D = 16
_B = 16384
_NC = 2
_NS = 16
_NW = _NC * _NS
_PER_W = _B // _NW
_R = 128
_NCHUNK = _PER_W // _R


def _fm_body(x_hbm, lin_hbm, emb_hbm, bias_hbm, out_hbm,
             xbuf, ebuf, lbuf, obuf, bbuf, sem_e, sem_l):
    wid = lax.axis_index("s") * _NC + lax.axis_index("c")
    base = wid * _PER_W

    pltpu.sync_copy(x_hbm.at[:, pl.ds(base, _PER_W)], xbuf)
    pltpu.sync_copy(bias_hbm, bbuf)

    def off_body(f, carry):
        off = f * _FIELD

        def g_body(g, carry2):
            sl = pl.ds(g * _D, _D)
            xbuf[f, sl] = xbuf[f, sl] + off
            return carry2

        return lax.fori_loop(0, _PER_W // _D, g_body, carry)

    lax.fori_loop(0, _F, off_body, 0)

    bval = bbuf[...]
    lane = lax.iota(jnp.int32, _D)

    def chunk_body(c, carry):
        col = pl.ds(c * _R, _R)

        def fire_body(f, carry2):
            idx = xbuf.at[f, col]
            pltpu.async_copy(emb_hbm.at[idx], ebuf.at[f], sem_e)
            pltpu.async_copy(lin_hbm.at[idx], lbuf.at[f], sem_l)
            return carry2

        lax.fori_loop(0, _F, fire_body, 0)

        def drain_body(f, carry2):
            pltpu.make_async_copy(emb_hbm.at[pl.ds(0, _R)], ebuf.at[f], sem_e).wait()
            pltpu.make_async_copy(lin_hbm.at[pl.ds(0, _R)], lbuf.at[f], sem_l).wait()
            return carry2

        lax.fori_loop(0, _F, drain_body, 0)

        def grp_body(g, carry2):
            fmvec = jnp.zeros((_D,), jnp.float32)
            for j in range(_D):
                r = g * _D + j
                s = ebuf[0, r]
                q = s * s
                for f in range(1, _F):
                    v = ebuf[f, r]
                    s = s + v
                    q = q + v * v
                fm = 0.5 * jnp.sum(s * s - q)
                fmvec = jnp.where(lane == j, fm, fmvec)
            sl = pl.ds(g * _D, _D)
            lin = lbuf[0, sl]
            for f in range(1, _F):
                lin = lin + lbuf[f, sl]
            z = lin + fmvec + bval
            obuf[pl.ds(c * _R + g * _D, _D)] = 1.0 / (1.0 + jnp.exp(-z))
            return carry2

        lax.fori_loop(0, _R // _D, grp_body, 0)
        return carry

    lax.fori_loop(0, _NCHUNK, chunk_body, 0)

    pltpu.sync_copy(obuf, out_hbm.at[pl.ds(base, _PER_W)])


@jax.jit
def kernel(x, linear_w, emb_w, bias):
    x_t = x.T.astype(jnp.int32)
    lin1d = linear_w.reshape(-1)
    bias_v = jnp.broadcast_to(bias.reshape(()), (_D,))

    mesh = plsc.VectorSubcoreMesh(
        core_axis_name="c", subcore_axis_name="s",
        num_cores=_NC, num_subcores=_NS)

    fm = pl.kernel(
        _fm_body,
        out_type=jax.ShapeDtypeStruct((_B,), jnp.float32),
        mesh=mesh,
        scratch_types=[
            pltpu.VMEM((_F, _PER_W), jnp.int32),
            pltpu.VMEM((_F, _R, _D), jnp.float32),
            pltpu.VMEM((_F, _R), jnp.float32),
            pltpu.VMEM((_PER_W,), jnp.float32),
            pltpu.VMEM((_D,), jnp.float32),
            pltpu.SemaphoreType.DMA,
            pltpu.SemaphoreType.DMA,
        ],
        compiler_params=pltpu.CompilerParams(
            needs_layout_passes=False, use_tc_tiling_on_sc=False),
    )
    return fm(x_t, lin1d, emb_w, bias_v)
